# baseline (speedup 1.0000x reference)
import functools

import jax
import jax.numpy as jnp
from jax.experimental import pallas as pl

N = 50000
D = 128
K = 16
R = 4
G_ = 2 * R
BR = 10000
ITERS = 2

CLS_BLK = 1000
REL_BLK = 1000


def _class_update_body(e_ref, m_ref, va_ref, vb_ref, wa_ref, wb_ref, o_ref):
    e = e_ref[...]
    m = m_ref[...]
    dn = (((1,), (1,)), ((), ()))
    gate_pre = (jax.lax.dot_general(e, va_ref[...], dn)
                + jax.lax.dot_general(m, vb_ref[...], dn))
    dir_pre = (jax.lax.dot_general(e, wa_ref[...], dn)
               + jax.lax.dot_general(m, wb_ref[...], dn))
    x = e + jax.nn.sigmoid(gate_pre) * jnp.maximum(dir_pre, 0.0)
    n = jnp.sqrt(jnp.sum(x * x, axis=1, keepdims=True))
    o_ref[...] = x / jnp.maximum(n, 1e-12)


def _class_update(e, m, class_V, class_W):
    va, vb = class_V[:, :D], class_V[:, D:]
    wa, wb = class_W[:, :D], class_W[:, D:]
    full = lambda s: pl.BlockSpec(s, lambda i: (0, 0))
    return pl.pallas_call(
        _class_update_body,
        grid=(N // CLS_BLK,),
        in_specs=[
            pl.BlockSpec((CLS_BLK, D), lambda i: (i, 0)),
            pl.BlockSpec((CLS_BLK, K), lambda i: (i, 0)),
            full((D, D)), full((D, K)), full((D, D)), full((D, K)),
        ],
        out_specs=pl.BlockSpec((CLS_BLK, D), lambda i: (i, 0)),
        out_shape=jax.ShapeDtypeStruct((N, D), jnp.float32),
    )(e, m, va, vb, wa, wb)


def _rel_body(es_ref, eo_ref, svs_ref, svo_ref, sws_ref, swo_ref, sw_ref,
              ovs_ref, ovo_ref, ows_ref, owo_ref, ow_ref, us_ref, uo_ref):
    es = es_ref[...]
    eo = eo_ref[...]
    dn = (((1,), (1,)), ((), ()))
    dg = jax.lax.dot_general
    gs = jax.nn.sigmoid(dg(es, svs_ref[...], dn) + dg(eo, svo_ref[...], dn))
    dot_s = jnp.sum(eo * sw_ref[...], axis=1, keepdims=True)
    dir_s = jnp.maximum(dg(es, sws_ref[...], dn) + dg(eo, swo_ref[...], dn)
                        + es * dot_s, 0.0)
    us_ref[...] = gs * dir_s
    go = jax.nn.sigmoid(dg(es, ovs_ref[...], dn) + dg(eo, ovo_ref[...], dn))
    dot_o = jnp.sum(eo * ow_ref[...], axis=1, keepdims=True)
    dir_o = jnp.maximum(dg(es, ows_ref[...], dn) + dg(eo, owo_ref[...], dn)
                        + es * dot_o, 0.0)
    uo_ref[...] = go * dir_o


def _rel_compute(es, eo, svs, svo, sws, swo, sw, ovs, ovo, ows, owo, ow):
    full = lambda: pl.BlockSpec((D, D), lambda i: (0, 0))
    vec = lambda: pl.BlockSpec((1, D), lambda i: (0, 0))
    blk = pl.BlockSpec((REL_BLK, D), lambda i: (i, 0))
    return pl.pallas_call(
        _rel_body,
        grid=(BR // REL_BLK,),
        in_specs=[blk, blk,
                  full(), full(), full(), full(), vec(),
                  full(), full(), full(), full(), vec()],
        out_specs=[blk, blk],
        out_shape=[jax.ShapeDtypeStruct((BR, D), jnp.float32),
                   jax.ShapeDtypeStruct((BR, D), jnp.float32)],
    )(es, eo, svs, svo, sws, swo, sw.reshape(1, D),
      ovs, ovo, ows, owo, ow.reshape(1, D))


def _norm_rows_body(x_ref, o_ref):
    x = x_ref[...]
    n = jnp.sqrt(jnp.sum(x * x, axis=1, keepdims=True))
    o_ref[...] = x / jnp.maximum(n, 1e-12)


def _norm_rows(x):
    blk = pl.BlockSpec((REL_BLK, D), lambda i: (i, 0))
    return pl.pallas_call(
        _norm_rows_body,
        grid=(x.shape[0] // REL_BLK,),
        in_specs=[blk],
        out_specs=blk,
        out_shape=jax.ShapeDtypeStruct(x.shape, jnp.float32),
    )(x)


def kernel(embeddings, memberships, subjects, objects, class_V, class_W,
           sub_Vs, sub_Vo, sub_Ws, sub_Wo, sub_w,
           obj_Vs, obj_Vo, obj_Ws, obj_Wo, obj_w):
    e = embeddings
    for _ in range(ITERS):
        e = _class_update(e, memberships, class_V, class_W)
        for g in range(G_):
            s_idx = subjects[g]
            o_idx = objects[g]
            es = jnp.take(e, s_idx, axis=0)
            eo = jnp.take(e, o_idx, axis=0)
            upd_s, upd_o = _rel_compute(
                es, eo,
                sub_Vs[g], sub_Vo[g], sub_Ws[g], sub_Wo[g], sub_w[g],
                obj_Vs[g], obj_Vo[g], obj_Ws[g], obj_Wo[g], obj_w[g])
            e = e.at[s_idx].add(upd_s)
            e = e.at[o_idx].add(upd_o)
            touched = jnp.concatenate([s_idx, o_idx])
            rows = jnp.take(e, touched, axis=0)
            e = e.at[touched].set(_norm_rows(rows))
    return e


# speedup vs baseline: 1.1431x; 1.1431x over previous
import functools

import jax
import jax.numpy as jnp
from jax import lax
from jax.experimental import pallas as pl
from jax.experimental.pallas import tpu as pltpu
from jax.experimental.pallas import tpu_sc as plsc

N = 50000
D = 128
K = 16
R = 4
G_ = 2 * R
BR = 10000
ITERS = 2

NP_ = 50176
NENT = 20480
NW = 32
EPW = NENT // NW
ICH = 128

CLS_BLK = 1024
REL_BLK = 1000


def _class_update_body(e_ref, m_ref, va_ref, vb_ref, wa_ref, wb_ref, o_ref):
    e = e_ref[...]
    m = m_ref[...]
    dn = (((1,), (1,)), ((), ()))
    gate_pre = (jax.lax.dot_general(e, va_ref[...], dn)
                + jax.lax.dot_general(m, vb_ref[...], dn))
    dir_pre = (jax.lax.dot_general(e, wa_ref[...], dn)
               + jax.lax.dot_general(m, wb_ref[...], dn))
    x = e + jax.nn.sigmoid(gate_pre) * jnp.maximum(dir_pre, 0.0)
    n = jnp.sqrt(jnp.sum(x * x, axis=1, keepdims=True))
    o_ref[...] = x / jnp.maximum(n, 1e-12)


def _class_update(e, m, class_V, class_W):
    va, vb = class_V[:, :D], class_V[:, D:]
    wa, wb = class_W[:, :D], class_W[:, D:]
    full = lambda s: pl.BlockSpec(s, lambda i: (0, 0))
    return pl.pallas_call(
        _class_update_body,
        grid=(NP_ // CLS_BLK,),
        in_specs=[
            pl.BlockSpec((CLS_BLK, D), lambda i: (i, 0)),
            pl.BlockSpec((CLS_BLK, K), lambda i: (i, 0)),
            full((D, D)), full((D, K)), full((D, D)), full((D, K)),
        ],
        out_specs=pl.BlockSpec((CLS_BLK, D), lambda i: (i, 0)),
        out_shape=jax.ShapeDtypeStruct((NP_, D), jnp.float32),
    )(e, m, va, vb, wa, wb)


def _rel_body(es_ref, eo_ref, svs_ref, svo_ref, sws_ref, swo_ref, sw_ref,
              ovs_ref, ovo_ref, ows_ref, owo_ref, ow_ref, us_ref, uo_ref):
    es = es_ref[...]
    eo = eo_ref[...]
    dn = (((1,), (1,)), ((), ()))
    dg = jax.lax.dot_general
    gs = jax.nn.sigmoid(dg(es, svs_ref[...], dn) + dg(eo, svo_ref[...], dn))
    dot_s = jnp.sum(eo * sw_ref[...], axis=1, keepdims=True)
    dir_s = jnp.maximum(dg(es, sws_ref[...], dn) + dg(eo, swo_ref[...], dn)
                        + es * dot_s, 0.0)
    us_ref[...] = gs * dir_s
    go = jax.nn.sigmoid(dg(es, ovs_ref[...], dn) + dg(eo, ovo_ref[...], dn))
    dot_o = jnp.sum(eo * ow_ref[...], axis=1, keepdims=True)
    dir_o = jnp.maximum(dg(es, ows_ref[...], dn) + dg(eo, owo_ref[...], dn)
                        + es * dot_o, 0.0)
    uo_ref[...] = go * dir_o


def _rel_compute(es, eo, svs, svo, sws, swo, sw, ovs, ovo, ows, owo, ow):
    full = lambda: pl.BlockSpec((D, D), lambda i: (0, 0))
    vec = lambda: pl.BlockSpec((1, D), lambda i: (0, 0))
    blk = pl.BlockSpec((REL_BLK, D), lambda i: (i, 0))
    return pl.pallas_call(
        _rel_body,
        grid=(BR // REL_BLK,),
        in_specs=[blk, blk,
                  full(), full(), full(), full(), vec(),
                  full(), full(), full(), full(), vec()],
        out_specs=[blk, blk],
        out_shape=[jax.ShapeDtypeStruct((BR, D), jnp.float32),
                   jax.ShapeDtypeStruct((BR, D), jnp.float32)],
    )(es, eo, svs, svo, sws, swo, sw.reshape(1, D),
      ovs, ovo, ows, owo, ow.reshape(1, D))


def _norm_rows_body(x_ref, o_ref):
    x = x_ref[...]
    n = jnp.sqrt(jnp.sum(x * x, axis=1, keepdims=True))
    o_ref[...] = x / jnp.maximum(n, 1e-12)


def _norm_rows(x):
    nb = 1024
    blk = pl.BlockSpec((nb, D), lambda i: (i, 0))
    return pl.pallas_call(
        _norm_rows_body,
        grid=(x.shape[0] // nb,),
        in_specs=[blk],
        out_specs=blk,
        out_shape=jax.ShapeDtypeStruct(x.shape, jnp.float32),
    )(x)


def _sc_gather(e, idx3d):
    kpw = NENT // ICH // NW
    mesh = plsc.VectorSubcoreMesh(core_axis_name="c", subcore_axis_name="s")

    @functools.partial(
        pl.kernel, mesh=mesh,
        out_type=jax.ShapeDtypeStruct((NENT, D), jnp.float32),
        scratch_types=[
            pltpu.VMEM((kpw, ICH), jnp.int32),
            pltpu.VMEM((EPW, D), jnp.float32),
            pltpu.SemaphoreType.DMA,
        ],
        name="sc_gather",
    )
    def k(e_hbm, idx_hbm, out_hbm, idx_v, rows_v, sem):
        wid = lax.axis_index("s") * 2 + lax.axis_index("c")
        pltpu.sync_copy(idx_hbm.at[wid], idx_v)
        cps = []
        for j in range(kpw):
            cps.append(pltpu.async_copy(
                e_hbm.at[idx_v.at[j]],
                rows_v.at[pl.ds(j * ICH, ICH)], sem))
        for cp in cps:
            cp.wait()
        pltpu.sync_copy(rows_v, out_hbm.at[pl.ds(wid * EPW, EPW)])

    return k(e, idx3d)


def kernel(embeddings, memberships, subjects, objects, class_V, class_W,
           sub_Vs, sub_Vo, sub_Ws, sub_Wo, sub_w,
           obj_Vs, obj_Vo, obj_Ws, obj_Wo, obj_w):
    e = jnp.pad(embeddings, ((0, NP_ - N), (0, 0)))
    m_pad = jnp.pad(memberships, ((0, NP_ - N), (0, 0)))
    npad = (NENT - 2 * BR) // 2
    pad_s = N + (jnp.arange(npad, dtype=jnp.int32) % (NP_ - N))
    pad_o = N + ((jnp.arange(npad, dtype=jnp.int32) + npad) % (NP_ - N))
    idx_all = jnp.concatenate(
        [subjects, jnp.tile(pad_s, (G_, 1)),
         objects, jnp.tile(pad_o, (G_, 1))], axis=1)
    idx3d = idx_all.reshape(G_, NW, NENT // ICH // NW, ICH)

    for _ in range(ITERS):
        e = _class_update(e, m_pad, class_V, class_W)
        for g in range(G_):
            rows = _sc_gather(e, idx3d[g])
            es = rows[:BR]
            eo = rows[BR + npad:2 * BR + npad]
            upd_s, upd_o = _rel_compute(
                es, eo,
                sub_Vs[g], sub_Vo[g], sub_Ws[g], sub_Wo[g], sub_w[g],
                obj_Vs[g], obj_Vo[g], obj_Ws[g], obj_Wo[g], obj_w[g])
            e = e.at[subjects[g]].add(upd_s)
            e = e.at[objects[g]].add(upd_o)
            rows2 = _sc_gather(e, idx3d[g])
            e = e.at[idx_all[g]].set(_norm_rows(rows2))
    return e[:N]
